# Initial kernel scaffold; baseline (speedup 1.0000x reference)
import jax
import jax.numpy as jnp
from jax.experimental import pallas as pl


def kernel(features, edge_index, W1, b1, W2, b2, W3, b3):
    raise NotImplementedError("write your pallas kernel here")


# speedup vs baseline: 4.7844x; 4.7844x over previous
import functools

import jax
import jax.numpy as jnp
from jax import lax
from jax.experimental import pallas as pl
from jax.experimental.pallas import tpu as pltpu
from jax.experimental.pallas import tpu_sc as plsc

_N = 10000
_E = 320000
_NC = 2
_NS = 16
_NW = _NC * _NS
_EPW = _E // _NW
_CHUNK = 80
_NCHUNK = _EPW // _CHUNK
_NA = _NS * 632
_RPS = _NA // _NS
_NP = _NS * 640
_DPS = _NP // _NS

_ROW_BLK = 1000


def _make_mesh():
    return plsc.VectorSubcoreMesh(core_axis_name="c", subcore_axis_name="s")


@functools.partial(
    pl.kernel,
    mesh=_make_mesh(),
    out_type=jax.ShapeDtypeStruct((_NC * _NP,), jnp.float32),
    scratch_types=[
        pltpu.VMEM((_CHUNK,), jnp.int32),
        pltpu.VMEM((_CHUNK,), jnp.float32),
        pltpu.VMEM((_DPS,), jnp.float32),
        pltpu.VMEM_SHARED((_NP,), jnp.float32),
    ],
)
def _deg_sc(dst_hbm, out_hbm, dst_v, ones_v, stage_v, dacc):
    c = lax.axis_index("c")
    s = lax.axis_index("s")
    wid = c * _NS + s
    r0 = s * _DPS

    def zbody(k, carry):
        stage_v[pl.ds(k * 16, 16)] = jnp.zeros((16,), jnp.float32)
        return carry

    lax.fori_loop(0, _DPS // 16, zbody, 0)
    pltpu.sync_copy(stage_v, dacc.at[pl.ds(r0, _DPS)])
    for k in range(_CHUNK // 16):
        ones_v[pl.ds(k * 16, 16)] = jnp.full((16,), 1.0, jnp.float32)
    plsc.subcore_barrier()

    def body(j, carry):
        base = wid * _EPW + j * _CHUNK
        pltpu.sync_copy(dst_hbm.at[pl.ds(base, _CHUNK)], dst_v)
        pltpu.sync_copy(ones_v, dacc.at[dst_v], add=True)
        return carry

    lax.fori_loop(0, _NCHUNK, body, 0)
    plsc.subcore_barrier()
    pltpu.sync_copy(dacc.at[pl.ds(r0, _DPS)], stage_v)
    pltpu.sync_copy(stage_v, out_hbm.at[pl.ds(c * _NP + r0, _DPS)])


def _make_agg(d):
    @functools.partial(
        pl.kernel,
        mesh=_make_mesh(),
        out_type=jax.ShapeDtypeStruct((_NC, _NA, d), jnp.float32),
        scratch_types=[
            pltpu.VMEM((_CHUNK,), jnp.int32),
            pltpu.VMEM((_CHUNK,), jnp.int32),
            pltpu.VMEM((_CHUNK, d), jnp.float32),
            pltpu.VMEM_SHARED((_NA, d), jnp.float32),
            pltpu.SemaphoreType.DMA,
        ],
    )
    def agg(t_hbm, src_hbm, dst_hbm, zeros_hbm, out_hbm, src_v, dst_v, rows_v, acc, sem):
        c = lax.axis_index("c")
        s = lax.axis_index("s")
        wid = c * _NS + s
        r0 = s * _RPS
        pltpu.sync_copy(zeros_hbm.at[pl.ds(r0, _RPS), :], acc.at[pl.ds(r0, _RPS), :])
        plsc.subcore_barrier()

        def body(j, carry):
            base = wid * _EPW + j * _CHUNK
            pltpu.sync_copy(src_hbm.at[pl.ds(base, _CHUNK)], src_v)
            pltpu.sync_copy(dst_hbm.at[pl.ds(base, _CHUNK)], dst_v)
            pltpu.async_copy(t_hbm.at[src_v], rows_v, sem).wait()
            pltpu.sync_copy(rows_v, acc.at[dst_v], add=True)
            return carry

        lax.fori_loop(0, _NCHUNK, body, 0)
        plsc.subcore_barrier()
        pltpu.sync_copy(acc.at[pl.ds(r0, _RPS), :], out_hbm.at[c, pl.ds(r0, _RPS), :])

    return agg


_agg128 = _make_agg(128)


def _mm_first(x, deg_p, w):
    def body(x_ref, deg_ref, w_ref, t_ref, d_ref):
        deg = deg_ref[0] + deg_ref[1]
        dn = jnp.where(deg > 0, lax.rsqrt(jnp.maximum(deg, 1.0)), 0.0)
        t_ref[...] = jnp.dot(x_ref[...] * dn, w_ref[...],
                             preferred_element_type=jnp.float32)
        d_ref[...] = dn

    din, dout = w.shape
    return pl.pallas_call(
        body,
        grid=(_N // _ROW_BLK,),
        in_specs=[
            pl.BlockSpec((_ROW_BLK, din), lambda i: (i, 0)),
            pl.BlockSpec((2, _ROW_BLK, 1), lambda i: (0, i, 0)),
            pl.BlockSpec((din, dout), lambda i: (0, 0)),
        ],
        out_specs=[
            pl.BlockSpec((_ROW_BLK, dout), lambda i: (i, 0)),
            pl.BlockSpec((_ROW_BLK, 1), lambda i: (i, 0)),
        ],
        out_shape=[
            jax.ShapeDtypeStruct((_N, dout), jnp.float32),
            jax.ShapeDtypeStruct((_N, 1), jnp.float32),
        ],
    )(x, deg_p, w)


def _mm_mid(agg_p, d, b, w):
    def body(a_ref, d_ref, b_ref, w_ref, o_ref):
        dn = d_ref[...]
        h = jnp.maximum((a_ref[0] + a_ref[1]) * dn + b_ref[...], 0.0) * dn
        o_ref[...] = jnp.dot(h, w_ref[...], preferred_element_type=jnp.float32)

    din, dout = w.shape
    return pl.pallas_call(
        body,
        grid=(_N // _ROW_BLK,),
        in_specs=[
            pl.BlockSpec((2, _ROW_BLK, din), lambda i: (0, i, 0)),
            pl.BlockSpec((_ROW_BLK, 1), lambda i: (i, 0)),
            pl.BlockSpec((1, din), lambda i: (0, 0)),
            pl.BlockSpec((din, dout), lambda i: (0, 0)),
        ],
        out_specs=pl.BlockSpec((_ROW_BLK, dout), lambda i: (i, 0)),
        out_shape=jax.ShapeDtypeStruct((_N, dout), jnp.float32),
    )(agg_p, d, b, w)


def _mm_last(agg_p, d, b):
    def body(a_ref, d_ref, b_ref, o_ref):
        o_ref[...] = (a_ref[0] + a_ref[1]) * d_ref[...] + b_ref[...]

    dout = agg_p.shape[-1]
    return pl.pallas_call(
        body,
        grid=(_N // _ROW_BLK,),
        in_specs=[
            pl.BlockSpec((2, _ROW_BLK, dout), lambda i: (0, i, 0)),
            pl.BlockSpec((_ROW_BLK, 1), lambda i: (i, 0)),
            pl.BlockSpec((1, dout), lambda i: (0, 0)),
        ],
        out_specs=pl.BlockSpec((_ROW_BLK, dout), lambda i: (i, 0)),
        out_shape=jax.ShapeDtypeStruct((_N, dout), jnp.float32),
    )(agg_p, d, b)


def kernel(features, edge_index, W1, b1, W2, b2, W3, b3):
    zeros128 = jnp.zeros((_NA, 128), jnp.float32)
    src_ids = edge_index[0]
    dst_ids = edge_index[1]
    deg_p = _deg_sc(dst_ids)
    deg_p = deg_p.reshape(_NC, _NP)[:, :_N].reshape(_NC, _N, 1)

    t1, d = _mm_first(features, deg_p, W1)
    agg1 = _agg128(t1, src_ids, dst_ids, zeros128)[:, :_N]
    t2 = _mm_mid(agg1, d, b1.reshape(1, -1), W2)
    agg2 = _agg128(t2, src_ids, dst_ids, zeros128)[:, :_N]
    w3p = jnp.pad(W3, ((0, 0), (0, 64)))
    t3 = _mm_mid(agg2, d, b2.reshape(1, -1), w3p)
    agg3 = _agg128(t3, src_ids, dst_ids, zeros128)[:, :_N, :64]
    out = _mm_last(agg3, d, b3.reshape(1, -1))
    return out
